# Initial kernel scaffold; baseline (speedup 1.0000x reference)
import jax
import jax.numpy as jnp
from jax.experimental import pallas as pl


def kernel(image, affine_transforms):
    raise NotImplementedError("write your pallas kernel here")


# speedup vs baseline: 1.1448x; 1.1448x over previous
import functools

import jax
import jax.numpy as jnp
from jax import lax
from jax.experimental import pallas as pl
from jax.experimental.pallas import tpu as pltpu
from jax.experimental.pallas import tpu_sc as plsc

OUT_H, OUT_W = 224, 224
NC, NS, L = 2, 16, 16
NW = NC * NS
CHUNK = 64


def _grid_coords(affine, B, H, W):
    xl = jnp.linspace(-1.0, 1.0, OUT_W)
    yl = jnp.linspace(-1.0, 1.0, OUT_H)
    X, Y = jnp.meshgrid(xl, yl)
    grid = jnp.concatenate(
        [X.flatten(), Y.flatten(), jnp.ones(OUT_H * OUT_W, jnp.float32)], 0)
    grids = jnp.tile(grid, (B,)).reshape(B, 3, OUT_H * OUT_W)
    aff = affine.reshape(B, 2, 3)
    grids = jnp.einsum('bij,bjk->bik', aff, grids)
    x = grids[:, 0, :].reshape(-1)
    y = grids[:, 1, :].reshape(-1)
    x = 0.5 * (x + 1.0) * jnp.float32(W)
    y = 0.5 * (y + 1.0) * jnp.float32(H)
    return x, y


@functools.cache
def _make_sc_kernel(N, H, W, C):
    per_w = N // NW
    n_chunks = per_w // CHUNK
    mesh = plsc.VectorSubcoreMesh(core_axis_name="c", subcore_axis_name="s")

    @functools.partial(
        pl.kernel, mesh=mesh,
        compiler_params=pltpu.CompilerParams(use_tc_tiling_on_sc=False),
        out_type=jax.ShapeDtypeStruct((N, C), jnp.float32),
        scratch_types=[
            pltpu.VMEM((CHUNK,), jnp.float32),
            pltpu.VMEM((CHUNK,), jnp.float32),
            pltpu.VMEM((CHUNK,), jnp.int32),
            pltpu.VMEM((CHUNK,), jnp.int32),
            pltpu.VMEM((CHUNK,), jnp.int32),
            pltpu.VMEM((CHUNK,), jnp.int32),
            pltpu.VMEM((CHUNK,), jnp.float32),
            pltpu.VMEM((CHUNK,), jnp.float32),
            pltpu.VMEM((CHUNK,), jnp.float32),
            pltpu.VMEM((CHUNK,), jnp.float32),
            pltpu.VMEM((CHUNK, C), jnp.float32),
            pltpu.VMEM((CHUNK, C), jnp.float32),
            pltpu.VMEM((CHUNK, C), jnp.float32),
            pltpu.VMEM((CHUNK, C), jnp.float32),
            pltpu.VMEM((CHUNK, C), jnp.float32),
            pltpu.SemaphoreType.DMA,
        ],
    )
    def k(x_hbm, y_hbm, tbl_hbm, out_hbm,
          xv, yv, ia, ib, ic, idd, wa, wb, wc, wd,
          ra, rb, rc, rd, ob, sem):
        wid = lax.axis_index("s") * NC + lax.axis_index("c")
        img_base = (wid * per_w) // (H * W) * (H * W)

        def chunk_body(ci, carry):
            start = wid * per_w + ci * CHUNK
            pltpu.sync_copy(x_hbm.at[pl.ds(start, CHUNK)], xv)
            pltpu.sync_copy(y_hbm.at[pl.ds(start, CHUNK)], yv)
            for v in range(CHUNK // L):
                sl = pl.ds(v * L, L)
                x = xv[sl]
                y = yv[sl]
                x_min = x.astype(jnp.int32)
                y_min = y.astype(jnp.int32)
                x_max = jnp.clip(x_min + 1, 0, W - 1)
                y_max = jnp.clip(y_min + 1, 0, H - 1)
                x_min = jnp.clip(x_min, 0, W - 1)
                y_min = jnp.clip(y_min, 0, H - 1)
                xmf = x_min.astype(jnp.float32)
                xMf = x_max.astype(jnp.float32)
                ymf = y_min.astype(jnp.float32)
                yMf = y_max.astype(jnp.float32)
                rmin = img_base + y_min * W
                rmax = img_base + y_max * W
                ia[sl] = rmin + x_min
                ib[sl] = rmax + x_min
                ic[sl] = rmin + x_max
                idd[sl] = rmax + x_max
                wa[sl] = (xMf - x) * (yMf - y)
                wb[sl] = (xMf - x) * (y - ymf)
                wc[sl] = (x - xmf) * (yMf - y)
                wd[sl] = (x - xmf) * (y - ymf)
            cpa = pltpu.async_copy(tbl_hbm.at[ia], ra, sem)
            cpb = pltpu.async_copy(tbl_hbm.at[ib], rb, sem)
            cpc = pltpu.async_copy(tbl_hbm.at[ic], rc, sem)
            cpd = pltpu.async_copy(tbl_hbm.at[idd], rd, sem)
            cpa.wait()
            cpb.wait()
            cpc.wait()
            cpd.wait()

            def grp_body(g, c2):
                base16 = g * L
                va = wa[pl.ds(base16, L)]
                vb = wb[pl.ds(base16, L)]
                vc = wc[pl.ds(base16, L)]
                vd = wd[pl.ds(base16, L)]
                for j in range(L):
                    p = base16 + j
                    a = va[j]
                    b = vb[j]
                    c = vc[j]
                    d = vd[j]
                    for ch in range(C // L):
                        s2 = pl.ds(ch * L, L)
                        ob[p, s2] = (ra[p, s2] * a + rb[p, s2] * b
                                     + rc[p, s2] * c + rd[p, s2] * d)
                return c2

            lax.fori_loop(0, CHUNK // L, grp_body, 0)
            pltpu.sync_copy(ob, out_hbm.at[pl.ds(start, CHUNK)])
            return carry

        lax.fori_loop(0, n_chunks, chunk_body, 0)

    return k


def kernel(image, affine_transforms):
    B, H, W, C = image.shape
    N = B * OUT_H * OUT_W
    x, y = _grid_coords(affine_transforms, B, H, W)
    flat = image.reshape(-1, C).astype(jnp.float32)
    out = _make_sc_kernel(N, H, W, C)(x, y, flat)
    return out.reshape(B, OUT_H, OUT_W, C)
